# Initial kernel scaffold; baseline (speedup 1.0000x reference)
import functools

import jax
import jax.numpy as jnp
from jax import lax
from jax.experimental import pallas as pl
from jax.experimental.pallas import tpu as pltpu
from jax.experimental.pallas import tpu_sc as plsc

F32 = jnp.float32

_N = 10000
_E = 320000
_D = 128
_NC = 2
_NS = 16
_L = 16
_NW = _NC * _NS
_EPT = _E // _NW
_CH = 80
_NCHUNK = _EPT // _CH
_RPT = _N // _NS
_ZR = 125
_CW = 16

_mesh = plsc.VectorSubcoreMesh(
    core_axis_name="c", subcore_axis_name="s", num_cores=_NC, num_subcores=_NS
)


def _agg_body(x_hbm, src_hbm, dst_hbm, agg_out, cnt_out,
              src_v, dst_v, rows_v, ones_v, zrow_v, zcnt_v,
              agg_sh, cnt_sh, sem):
    cid = lax.axis_index("c")
    sid = lax.axis_index("s")
    wid = cid * _NS + sid

    def fill_ones(r, carry):
        ones_v[r] = jnp.full((_L,), 1.0, F32)
        return carry

    lax.fori_loop(0, _CH, fill_ones, 0)

    def zero_zrow(r, carry):
        for c in range(_D // _L):
            zrow_v[r, pl.ds(c * _L, _L)] = jnp.zeros((_L,), F32)
        return carry

    lax.fori_loop(0, _ZR, zero_zrow, 0)

    def zero_zcnt(r, carry):
        zcnt_v[r] = jnp.zeros((_L,), F32)
        return carry

    lax.fori_loop(0, _RPT, zero_zcnt, 0)

    for k in range(_RPT // _ZR):
        pltpu.sync_copy(zrow_v, agg_sh.at[pl.ds(sid * _RPT + k * _ZR, _ZR)])
    pltpu.sync_copy(zcnt_v, cnt_sh.at[pl.ds(sid * _RPT, _RPT)])

    pltpu.sync_copy(src_hbm.at[wid], src_v)
    pltpu.sync_copy(dst_hbm.at[wid], dst_v)

    plsc.subcore_barrier()

    def step(j, carry):
        pltpu.async_copy(x_hbm.at[src_v.at[j]], rows_v, sem).wait()
        pltpu.sync_copy(rows_v, agg_sh.at[dst_v.at[j]], add=True)
        pltpu.sync_copy(ones_v, cnt_sh.at[dst_v.at[j]], add=True)
        return carry

    lax.fori_loop(0, _NCHUNK, step, 0)

    plsc.subcore_barrier()

    pltpu.sync_copy(agg_sh.at[pl.ds(sid * _RPT, _RPT)],
                    agg_out.at[pl.ds(cid * _N + sid * _RPT, _RPT)])
    pltpu.sync_copy(cnt_sh.at[pl.ds(sid * _RPT, _RPT)],
                    cnt_out.at[pl.ds(cid * _N + sid * _RPT, _RPT)])


_sc_agg = functools.partial(
    pl.kernel,
    out_type=(
        jax.ShapeDtypeStruct((_NC * _N, _D), F32),
        jax.ShapeDtypeStruct((_NC * _N, _CW), F32),
    ),
    mesh=_mesh,
    scratch_types=[
        pltpu.VMEM((_NCHUNK, _CH), jnp.int32),
        pltpu.VMEM((_NCHUNK, _CH), jnp.int32),
        pltpu.VMEM((_CH, _D), F32),
        pltpu.VMEM((_CH, _CW), F32),
        pltpu.VMEM((_ZR, _D), F32),
        pltpu.VMEM((_RPT, _CW), F32),
        pltpu.VMEM_SHARED((_N, _D), F32),
        pltpu.VMEM_SHARED((_N, _CW), F32),
        pltpu.SemaphoreType.DMA,
    ],
)(_agg_body)


_RB = 1000


def _dense1_body(agg0_ref, agg1_ref, cnt0_ref, cnt1_ref, x_ref,
                 wl_ref, wr_ref, b_ref, out_ref):
    cnt = cnt0_ref[:, 0:1] + cnt1_ref[:, 0:1]
    rinv = 1.0 / jnp.maximum(cnt, 1.0)
    agg = (agg0_ref[...] + agg1_ref[...]) * rinv
    h = jnp.dot(agg, wl_ref[...], preferred_element_type=F32)
    h = h + jnp.dot(x_ref[...], wr_ref[...], preferred_element_type=F32)
    h = h + b_ref[...]
    out_ref[...] = jnp.maximum(h, 0.0)


def _dense2_body(agg0_ref, agg1_ref, cnt0_ref, cnt1_ref, x_ref,
                 wl_ref, wr_ref, b_ref, out_ref):
    cnt = cnt0_ref[:, 0:1] + cnt1_ref[:, 0:1]
    rinv = 1.0 / jnp.maximum(cnt, 1.0)
    agg = (agg0_ref[...] + agg1_ref[...]) * rinv
    h = jnp.dot(agg, wl_ref[...], preferred_element_type=F32)
    h = h + jnp.dot(x_ref[...], wr_ref[...], preferred_element_type=F32)
    logits = h + b_ref[...]
    m = jnp.max(logits, axis=1, keepdims=True)
    lse = jnp.log(jnp.sum(jnp.exp(logits - m), axis=1, keepdims=True)) + m
    out_ref[...] = logits - lse


def _dense_call(body):
    row = lambda i: (i, 0)
    fix = lambda i: (0, 0)
    return pl.pallas_call(
        body,
        grid=(_N // _RB,),
        in_specs=[
            pl.BlockSpec((_RB, _D), row),
            pl.BlockSpec((_RB, _D), row),
            pl.BlockSpec((_RB, _CW), row),
            pl.BlockSpec((_RB, _CW), row),
            pl.BlockSpec((_RB, _D), row),
            pl.BlockSpec((_D, _D), fix),
            pl.BlockSpec((_D, _D), fix),
            pl.BlockSpec((1, _D), fix),
        ],
        out_specs=pl.BlockSpec((_RB, _D), row),
        out_shape=jax.ShapeDtypeStruct((_N, _D), F32),
    )


def kernel(x, edge_index, W1_l, b1_l, W1_r, W2_l, b2_l, W2_r):
    src = edge_index[0].reshape(_NW, _NCHUNK, _CH)
    dst = edge_index[1].reshape(_NW, _NCHUNK, _CH)

    agg1p, cntp = _sc_agg(x, src, dst)
    cnt0, cnt1 = cntp[:_N], cntp[_N:]

    h1 = _dense_call(_dense1_body)(
        agg1p[:_N], agg1p[_N:], cnt0, cnt1, x,
        W1_l.T, W1_r.T, b1_l.reshape(1, _D))

    agg2p, _ = _sc_agg(h1, src, dst)

    w2l = jnp.zeros((_D, _D), F32).at[:, :40].set(W2_l.T)
    w2r = jnp.zeros((_D, _D), F32).at[:, :40].set(W2_r.T)
    b2p = jnp.full((1, _D), -1e30, F32).at[0, :40].set(b2_l)

    out = _dense_call(_dense2_body)(
        agg2p[:_N], agg2p[_N:], cnt0, cnt1, h1, w2l, w2r, b2p)
    return out[:, :40]


# speedup vs baseline: 5.0037x; 5.0037x over previous
import functools

import jax
import jax.numpy as jnp
from jax import lax
from jax.experimental import pallas as pl
from jax.experimental.pallas import tpu as pltpu
from jax.experimental.pallas import tpu_sc as plsc

F32 = jnp.float32

_N = 10000
_E = 320000
_D = 128
_DH = 64
_NC = 2
_NS = 16
_L = 16
_EPT = _E // _NS
_CH = 80
_NCHUNK = _EPT // _CH
_NP = 10240
_RPT = _NP // _NS
_ZR = 128
_CW = 16

_mesh = plsc.VectorSubcoreMesh(
    core_axis_name="c", subcore_axis_name="s", num_cores=_NC, num_subcores=_NS
)


def _agg_body(x2_hbm, src_hbm, dst_hbm, agg_out, cnt_out,
              src_v, dst_v, rows_v, ones_v, zrow_v, zcnt_v,
              agg_sh, cnt_sh, sem):
    cid = lax.axis_index("c")
    sid = lax.axis_index("s")

    def fill_ones(r, carry):
        ones_v[r] = jnp.full((_L,), 1.0, F32)
        return carry

    lax.fori_loop(0, _CH, fill_ones, 0)

    def zero_zrow(r, carry):
        for c in range(_DH // _L):
            zrow_v[r, pl.ds(c * _L, _L)] = jnp.zeros((_L,), F32)
        return carry

    lax.fori_loop(0, _ZR, zero_zrow, 0)

    def zero_zcnt(r, carry):
        zcnt_v[r] = jnp.zeros((_L,), F32)
        return carry

    lax.fori_loop(0, _RPT, zero_zcnt, 0)

    for k in range(_RPT // _ZR):
        pltpu.sync_copy(zrow_v, agg_sh.at[pl.ds(sid * _RPT + k * _ZR, _ZR)])
    pltpu.sync_copy(zcnt_v, cnt_sh.at[pl.ds(sid * _RPT, _RPT)])

    pltpu.sync_copy(src_hbm.at[sid], src_v)
    pltpu.sync_copy(dst_hbm.at[sid], dst_v)
    off = cid * _N

    def adjust(i, carry):
        r = i // (_CH // _L)
        c = i % (_CH // _L)
        src_v[r, pl.ds(c * _L, _L)] = (
            src_v[r, pl.ds(c * _L, _L)] + off
        )
        return carry

    lax.fori_loop(0, _NCHUNK * (_CH // _L), adjust, 0)

    plsc.subcore_barrier()

    def step(j, carry):
        pltpu.async_copy(x2_hbm.at[src_v.at[j]], rows_v, sem).wait()
        pltpu.sync_copy(rows_v, agg_sh.at[dst_v.at[j]], add=True)

        @pl.when(cid == 0)
        def _():
            pltpu.sync_copy(ones_v, cnt_sh.at[dst_v.at[j]], add=True)

        return carry

    lax.fori_loop(0, _NCHUNK, step, 0)

    plsc.subcore_barrier()

    pltpu.sync_copy(agg_sh.at[pl.ds(sid * _RPT, _RPT)],
                    agg_out.at[pl.ds(cid * _NP + sid * _RPT, _RPT)])

    @pl.when(cid == 0)
    def _():
        pltpu.sync_copy(cnt_sh.at[pl.ds(sid * _RPT, _RPT)],
                        cnt_out.at[pl.ds(sid * _RPT, _RPT)])


_sc_agg = functools.partial(
    pl.kernel,
    out_type=(
        jax.ShapeDtypeStruct((_NC * _NP, _DH), F32),
        jax.ShapeDtypeStruct((_NP, _CW), F32),
    ),
    mesh=_mesh,
    compiler_params=pltpu.CompilerParams(use_tc_tiling_on_sc=False),
    scratch_types=[
        pltpu.VMEM((_NCHUNK, _CH), jnp.int32),
        pltpu.VMEM((_NCHUNK, _CH), jnp.int32),
        pltpu.VMEM((_CH, _DH), F32),
        pltpu.VMEM((_CH, _CW), F32),
        pltpu.VMEM((_ZR, _DH), F32),
        pltpu.VMEM((_RPT, _CW), F32),
        pltpu.VMEM_SHARED((_NP, _DH), F32),
        pltpu.VMEM_SHARED((_NP, _CW), F32),
        pltpu.SemaphoreType.DMA,
    ],
)(_agg_body)


def _split_cols(h):
    return jnp.swapaxes(h.reshape(_N, _NC, _DH), 0, 1).reshape(_NC * _N, _DH)


_RB = 1000


def _dense1_body(agglo_ref, agghi_ref, cnt_ref, x_ref,
                 wllo_ref, wlhi_ref, wr_ref, b_ref, out_ref):
    rinv = 1.0 / jnp.maximum(cnt_ref[:, 0:1], 1.0)
    h = jnp.dot(agglo_ref[...] * rinv, wllo_ref[...],
                preferred_element_type=F32)
    h = h + jnp.dot(agghi_ref[...] * rinv, wlhi_ref[...],
                    preferred_element_type=F32)
    h = h + jnp.dot(x_ref[...], wr_ref[...], preferred_element_type=F32)
    h = h + b_ref[...]
    out_ref[...] = jnp.maximum(h, 0.0)


def _dense2_body(agglo_ref, agghi_ref, cnt_ref, x_ref,
                 wllo_ref, wlhi_ref, wr_ref, b_ref, out_ref):
    rinv = 1.0 / jnp.maximum(cnt_ref[:, 0:1], 1.0)
    h = jnp.dot(agglo_ref[...] * rinv, wllo_ref[...],
                preferred_element_type=F32)
    h = h + jnp.dot(agghi_ref[...] * rinv, wlhi_ref[...],
                    preferred_element_type=F32)
    h = h + jnp.dot(x_ref[...], wr_ref[...], preferred_element_type=F32)
    logits = h + b_ref[...]
    m = jnp.max(logits, axis=1, keepdims=True)
    lse = jnp.log(jnp.sum(jnp.exp(logits - m), axis=1, keepdims=True)) + m
    out_ref[...] = logits - lse


def _dense_call(body):
    row = lambda i: (i, 0)
    fix = lambda i: (0, 0)
    return pl.pallas_call(
        body,
        grid=(_N // _RB,),
        in_specs=[
            pl.BlockSpec((_RB, _DH), row),
            pl.BlockSpec((_RB, _DH), row),
            pl.BlockSpec((_RB, _CW), row),
            pl.BlockSpec((_RB, _D), row),
            pl.BlockSpec((_DH, _D), fix),
            pl.BlockSpec((_DH, _D), fix),
            pl.BlockSpec((_D, _D), fix),
            pl.BlockSpec((1, _D), fix),
        ],
        out_specs=pl.BlockSpec((_RB, _D), row),
        out_shape=jax.ShapeDtypeStruct((_N, _D), F32),
    )


def kernel(x, edge_index, W1_l, b1_l, W1_r, W2_l, b2_l, W2_r):
    src = edge_index[0].reshape(_NS, _NCHUNK, _CH)
    dst = edge_index[1].reshape(_NS, _NCHUNK, _CH)

    agg1, cnt = _sc_agg(_split_cols(x), src, dst)
    cnt = cnt[:_N]

    w1t = W1_l.T
    h1 = _dense_call(_dense1_body)(
        agg1[:_N], agg1[_NP:_NP + _N], cnt, x,
        w1t[:_DH], w1t[_DH:], W1_r.T, b1_l.reshape(1, _D))

    agg2, _ = _sc_agg(_split_cols(h1), src, dst)

    w2l = jnp.zeros((_D, _D), F32).at[:, :40].set(W2_l.T)
    w2r = jnp.zeros((_D, _D), F32).at[:, :40].set(W2_r.T)
    b2p = jnp.full((1, _D), -1e30, F32).at[0, :40].set(b2_l)

    out = _dense_call(_dense2_body)(
        agg2[:_N], agg2[_NP:_NP + _N], cnt, h1,
        w2l[:_DH], w2l[_DH:], w2r, b2p)
    return out[:, :40]


# speedup vs baseline: 8.2528x; 1.6493x over previous
import functools

import jax
import jax.numpy as jnp
from jax import lax
from jax.experimental import pallas as pl
from jax.experimental.pallas import tpu as pltpu
from jax.experimental.pallas import tpu_sc as plsc

F32 = jnp.float32

_N = 10000
_E = 320000
_D = 128
_DH = 64
_NC = 2
_NS = 16
_L = 16
_EPT = _E // _NS
_CH = 80
_NCHUNK = _EPT // _CH
_NP = 10240
_RPT = _NP // _NS
_ZR = 128
_CW = 16

_mesh = plsc.VectorSubcoreMesh(
    core_axis_name="c", subcore_axis_name="s", num_cores=_NC, num_subcores=_NS
)


def _make_agg_body(with_counts):
    def _agg_body(x2_hbm, src_hbm, dst_hbm, *rest):
        if with_counts:
            (agg_out, cnt_out, src_v, dst_v, rows_v, ones_v, zrow_v,
             zcnt_v, agg_sh, cnt_sh, sem0, sem1) = rest
        else:
            (agg_out, src_v, dst_v, rows_v, zrow_v,
             agg_sh, sem0, sem1) = rest
        cid = lax.axis_index("c")
        sid = lax.axis_index("s")

        def zero_zrow(r, carry):
            for c in range(_DH // _L):
                zrow_v[r, pl.ds(c * _L, _L)] = jnp.zeros((_L,), F32)
            return carry

        lax.fori_loop(0, _ZR, zero_zrow, 0)

        for k in range(_RPT // _ZR):
            pltpu.sync_copy(zrow_v,
                            agg_sh.at[pl.ds(sid * _RPT + k * _ZR, _ZR)])

        if with_counts:
            def fill_ones(r, carry):
                ones_v[r] = jnp.full((_L,), 1.0, F32)
                return carry

            lax.fori_loop(0, _CH, fill_ones, 0)

            def zero_zcnt(r, carry):
                zcnt_v[r] = jnp.zeros((_L,), F32)
                return carry

            lax.fori_loop(0, _RPT, zero_zcnt, 0)
            pltpu.sync_copy(zcnt_v, cnt_sh.at[pl.ds(sid * _RPT, _RPT)])

        pltpu.sync_copy(src_hbm.at[sid], src_v)
        pltpu.sync_copy(dst_hbm.at[sid], dst_v)
        off = cid * _N

        def adjust(i, carry):
            r = i // (_CH // _L)
            c = i % (_CH // _L)
            src_v[r, pl.ds(c * _L, _L)] = (
                src_v[r, pl.ds(c * _L, _L)] + off
            )
            return carry

        lax.fori_loop(0, _NCHUNK * (_CH // _L), adjust, 0)

        plsc.subcore_barrier()

        sems = (sem0, sem1)

        def gather(j, b):
            return pltpu.async_copy(
                x2_hbm.at[src_v.at[j]], rows_v.at[b], sems[b])

        def consume(j, b):
            pltpu.make_async_copy(
                x2_hbm.at[src_v.at[j]], rows_v.at[b], sems[b]).wait()
            pltpu.sync_copy(rows_v.at[b], agg_sh.at[dst_v.at[j]], add=True)
            if with_counts:
                @pl.when(cid == 0)
                def _():
                    pltpu.sync_copy(ones_v, cnt_sh.at[dst_v.at[j]],
                                    add=True)

        gather(0, 0)
        gather(1, 1)

        def step(j2, carry):
            for b in range(2):
                j = 2 * j2 + b
                consume(j, b)
                gather(j + 2, b)
            return carry

        lax.fori_loop(0, _NCHUNK // 2 - 1, step, 0)
        for b in range(2):
            consume(_NCHUNK - 2 + b, b)

        plsc.subcore_barrier()

        pltpu.sync_copy(agg_sh.at[pl.ds(sid * _RPT, _RPT)],
                        agg_out.at[pl.ds(cid * _NP + sid * _RPT, _RPT)])

        if with_counts:
            @pl.when(cid == 0)
            def _():
                pltpu.sync_copy(cnt_sh.at[pl.ds(sid * _RPT, _RPT)],
                                cnt_out.at[pl.ds(sid * _RPT, _RPT)])

    return _agg_body


def _make_agg_kernel(with_counts):
    out_type = [jax.ShapeDtypeStruct((_NC * _NP, _DH), F32)]
    scratch = [
        pltpu.VMEM((_NCHUNK, _CH), jnp.int32),
        pltpu.VMEM((_NCHUNK, _CH), jnp.int32),
        pltpu.VMEM((2, _CH, _DH), F32),
    ]
    if with_counts:
        out_type.append(jax.ShapeDtypeStruct((_NP, _CW), F32))
        scratch.append(pltpu.VMEM((_CH, _CW), F32))
    scratch.append(pltpu.VMEM((_ZR, _DH), F32))
    if with_counts:
        scratch.append(pltpu.VMEM((_RPT, _CW), F32))
    scratch.append(pltpu.VMEM_SHARED((_NP, _DH), F32))
    if with_counts:
        scratch.append(pltpu.VMEM_SHARED((_NP, _CW), F32))
    scratch += [pltpu.SemaphoreType.DMA, pltpu.SemaphoreType.DMA]
    return pl.kernel(
        _make_agg_body(with_counts),
        out_type=tuple(out_type) if with_counts else out_type[0],
        mesh=_mesh,
        compiler_params=pltpu.CompilerParams(use_tc_tiling_on_sc=False),
        scratch_types=scratch,
    )


_sc_agg_cnt = _make_agg_kernel(True)
_sc_agg = _make_agg_kernel(False)


def _split_cols(h):
    return jnp.swapaxes(h.reshape(_N, _NC, _DH), 0, 1).reshape(_NC * _N, _DH)


_RB = 1000


def _dense1_body(agglo_ref, agghi_ref, cnt_ref, x_ref,
                 wllo_ref, wlhi_ref, wr_ref, b_ref, out_ref):
    rinv = 1.0 / jnp.maximum(cnt_ref[:, 0:1], 1.0)
    h = jnp.dot(agglo_ref[...] * rinv, wllo_ref[...],
                preferred_element_type=F32)
    h = h + jnp.dot(agghi_ref[...] * rinv, wlhi_ref[...],
                    preferred_element_type=F32)
    h = h + jnp.dot(x_ref[...], wr_ref[...], preferred_element_type=F32)
    h = h + b_ref[...]
    out_ref[...] = jnp.maximum(h, 0.0)


def _dense2_body(agglo_ref, agghi_ref, cnt_ref, x_ref,
                 wllo_ref, wlhi_ref, wr_ref, b_ref, out_ref):
    rinv = 1.0 / jnp.maximum(cnt_ref[:, 0:1], 1.0)
    h = jnp.dot(agglo_ref[...] * rinv, wllo_ref[...],
                preferred_element_type=F32)
    h = h + jnp.dot(agghi_ref[...] * rinv, wlhi_ref[...],
                    preferred_element_type=F32)
    h = h + jnp.dot(x_ref[...], wr_ref[...], preferred_element_type=F32)
    logits = h + b_ref[...]
    m = jnp.max(logits, axis=1, keepdims=True)
    lse = jnp.log(jnp.sum(jnp.exp(logits - m), axis=1, keepdims=True)) + m
    out_ref[...] = logits - lse


def _dense_call(body):
    row = lambda i: (i, 0)
    fix = lambda i: (0, 0)
    return pl.pallas_call(
        body,
        grid=(_N // _RB,),
        in_specs=[
            pl.BlockSpec((_RB, _DH), row),
            pl.BlockSpec((_RB, _DH), row),
            pl.BlockSpec((_RB, _CW), row),
            pl.BlockSpec((_RB, _D), row),
            pl.BlockSpec((_DH, _D), fix),
            pl.BlockSpec((_DH, _D), fix),
            pl.BlockSpec((_D, _D), fix),
            pl.BlockSpec((1, _D), fix),
        ],
        out_specs=pl.BlockSpec((_RB, _D), row),
        out_shape=jax.ShapeDtypeStruct((_N, _D), F32),
    )


def kernel(x, edge_index, W1_l, b1_l, W1_r, W2_l, b2_l, W2_r):
    src = edge_index[0].reshape(_NS, _NCHUNK, _CH)
    dst = edge_index[1].reshape(_NS, _NCHUNK, _CH)

    agg1, cnt = _sc_agg_cnt(_split_cols(x), src, dst)
    cnt = cnt[:_N]

    w1t = W1_l.T
    h1 = _dense_call(_dense1_body)(
        agg1[:_N], agg1[_NP:_NP + _N], cnt, x,
        w1t[:_DH], w1t[_DH:], W1_r.T, b1_l.reshape(1, _D))

    agg2 = _sc_agg(_split_cols(h1), src, dst)

    w2l = jnp.zeros((_D, _D), F32).at[:, :40].set(W2_l.T)
    w2r = jnp.zeros((_D, _D), F32).at[:, :40].set(W2_r.T)
    b2p = jnp.full((1, _D), -1e30, F32).at[0, :40].set(b2_l)

    out = _dense_call(_dense2_body)(
        agg2[:_N], agg2[_NP:_NP + _N], cnt, h1,
        w2l[:_DH], w2l[_DH:], w2r, b2p)
    return out[:, :40]


# speedup vs baseline: 14.4035x; 1.7453x over previous
import jax
import jax.numpy as jnp
from jax import lax
from jax.experimental import pallas as pl
from jax.experimental.pallas import tpu as pltpu
from jax.experimental.pallas import tpu_sc as plsc

F32 = jnp.float32

_N = 10000
_E = 320000
_D = 128
_DH = 64
_NC = 2
_NS = 16
_L = 16
_EPT = _E // _NS
_CH = 80
_NCHUNK = _EPT // _CH
_RPT = _N // _NS
_ZR = 125
_CW = 16
_NSS = 2
_RB = 1000

_mesh = plsc.VectorSubcoreMesh(
    core_axis_name="c", subcore_axis_name="s", num_cores=_NC, num_subcores=_NS
)


def _make_agg_body(with_counts, _NBUF):
    def _agg_body(x2_hbm, edges_hbm, *rest):
        if with_counts:
            (agg_out, cnt_out, src_v, dst_v, rows_v, zrow_v,
             cnt_t, agg_sh, *sems) = rest
        else:
            (agg_out, src_v, dst_v, rows_v, zrow_v,
             agg_sh, *sems) = rest
        gsem = tuple(sems[:_NBUF])
        ssem = tuple(sems[_NBUF:_NBUF + _NSS])
        cid = lax.axis_index("c")
        sid = lax.axis_index("s")

        def zero_zrow(r, carry):
            for c in range(_DH // _L):
                zrow_v[r, pl.ds(c * _L, _L)] = jnp.zeros((_L,), F32)
            return carry

        lax.fori_loop(0, _ZR, zero_zrow, 0)

        for k in range(_RPT // _ZR):
            pltpu.sync_copy(zrow_v,
                            agg_sh.at[pl.ds(sid * _RPT + k * _ZR, _ZR)])

        if with_counts:
            def zero_cnt(r, carry):
                cnt_t[pl.ds(r * _L, _L)] = jnp.zeros((_L,), F32)
                return carry

            lax.fori_loop(0, _N // _L, zero_cnt, 0)

        pltpu.sync_copy(edges_hbm.at[0, sid], src_v)
        pltpu.sync_copy(edges_hbm.at[1, sid], dst_v)

        def adjust(i, carry):
            r = i // (_CH // _L)
            c = i % (_CH // _L)
            src_v[r, pl.ds(c * _L, _L)] = (
                src_v[r, pl.ds(c * _L, _L)] * 2 + cid
            )
            return carry

        lax.fori_loop(0, _NCHUNK * (_CH // _L), adjust, 0)

        plsc.subcore_barrier()

        def issue_gather(j, b):
            pltpu.async_copy(x2_hbm.at[src_v.at[j]], rows_v.at[b], gsem[b])

        def wait_gather(b):
            pltpu.make_async_copy(
                x2_hbm.at[src_v.at[0]], rows_v.at[b], gsem[b]).wait()

        def issue_scatter(j, b, k):
            pltpu.async_copy(rows_v.at[b], agg_sh.at[dst_v.at[j]],
                             ssem[k], add=True)

        def wait_scatter(k):
            pltpu.make_async_copy(
                rows_v.at[0], agg_sh.at[dst_v.at[0]], ssem[k]).wait()

        _ones16 = jnp.full((_L,), 1.0, F32)

        def count_chunk(j):
            for c in range(_CH // _L):
                idx = dst_v[j, pl.ds(c * _L, _L)]
                plsc.addupdate_scatter(cnt_t, [idx], _ones16)

        for b in range(_NBUF - 2):
            issue_gather(b, b)
        for k in range(_NSS):
            pltpu.async_copy(zrow_v.at[pl.ds(0, _CH)],
                             agg_sh.at[dst_v.at[0]], ssem[k], add=True)

        def step(j6, carry):
            for u in range(_NBUF):
                j = _NBUF * j6 + u
                wait_gather(u)
                wait_scatter(u % _NSS)
                issue_scatter(j, u, u % _NSS)
                if with_counts:
                    @pl.when(cid == 0)
                    def _():
                        count_chunk(j)
                issue_gather(j + _NBUF - 2, (u + _NBUF - 2) % _NBUF)
            return carry

        _MAIN = _NCHUNK - (_NBUF - 2)
        lax.fori_loop(0, _MAIN // _NBUF, step, 0)

        for u in range(_NBUF - 2):
            j = _MAIN + u
            b = j % _NBUF
            wait_gather(b)
            wait_scatter(j % _NSS)
            issue_scatter(j, b, j % _NSS)
            if with_counts:
                @pl.when(cid == 0)
                def _():
                    count_chunk(j)
        for k in range(_NSS):
            wait_scatter((_NCHUNK - _NSS + k) % _NSS)

        plsc.subcore_barrier()

        row0 = sid * _RPT
        pltpu.sync_copy(agg_sh.at[pl.ds(row0, _RPT)],
                        agg_out.at[pl.ds(cid * _N + row0, _RPT)])

        if with_counts:
            @pl.when(cid == 0)
            def _():
                for k in range(_N // _RB):
                    pltpu.sync_copy(cnt_t.at[pl.ds(k * _RB, _RB)],
                                    cnt_out.at[k, sid])

    return _agg_body


def _make_agg_kernel(with_counts, _NBUF):
    out_type = [jax.ShapeDtypeStruct((_NC * _N, _DH), F32)]
    scratch = [
        pltpu.VMEM((_NCHUNK, _CH), jnp.int32),
        pltpu.VMEM((_NCHUNK, _CH), jnp.int32),
        pltpu.VMEM((_NBUF, _CH, _DH), F32),
    ]
    if with_counts:
        out_type.append(jax.ShapeDtypeStruct((_N // _RB, _NS, _RB), F32))
    scratch.append(pltpu.VMEM((_ZR, _DH), F32))
    if with_counts:
        scratch.append(pltpu.VMEM((_N,), F32))
    scratch.append(pltpu.VMEM_SHARED((_N, _DH), F32))
    scratch += [pltpu.SemaphoreType.DMA] * (_NBUF + _NSS)
    return pl.kernel(
        _make_agg_body(with_counts, _NBUF),
        out_type=tuple(out_type) if with_counts else out_type[0],
        mesh=_mesh,
        compiler_params=pltpu.CompilerParams(use_tc_tiling_on_sc=False,
                                            needs_layout_passes=False),
        scratch_types=scratch,
    )


_sc_agg_cnt = _make_agg_kernel(True, 6)
_sc_agg = _make_agg_kernel(False, 6)


def _dotT(a, w):
    return lax.dot_general(a, w, (((1,), (1,)), ((), ())),
                           preferred_element_type=F32)


def _dense1_body(agglo_ref, agghi_ref, cnt_ref, x_ref,
                 wl_ref, wr_ref, b_ref, out_ref):
    cnt = jnp.sum(cnt_ref[0], axis=0)[:, None]
    rinv = 1.0 / jnp.maximum(cnt, 1.0)
    wl = wl_ref[...]
    h = _dotT(agglo_ref[...] * rinv, wl[:, :_DH])
    h = h + _dotT(agghi_ref[...] * rinv, wl[:, _DH:])
    h = h + _dotT(x_ref[...], wr_ref[...])
    h = h + b_ref[...]
    out_ref[...] = jnp.maximum(h, 0.0)


def _dense2_body(agglo_ref, agghi_ref, cnt_ref, x_ref,
                 wl_ref, wr_ref, b_ref, out_ref):
    cnt = jnp.sum(cnt_ref[0], axis=0)[:, None]
    rinv = 1.0 / jnp.maximum(cnt, 1.0)
    wl = wl_ref[...]
    h = _dotT(agglo_ref[...] * rinv, wl[:, :_DH])
    h = h + _dotT(agghi_ref[...] * rinv, wl[:, _DH:])
    h = h + _dotT(x_ref[...], wr_ref[...])
    logits = h + b_ref[...]
    m = jnp.max(logits, axis=1, keepdims=True)
    lse = jnp.log(jnp.sum(jnp.exp(logits - m), axis=1, keepdims=True)) + m
    out_ref[...] = logits - lse


def _dense_call(body, dout):
    row = lambda i: (i, 0)
    hi = lambda i: (_N // _RB + i, 0)
    fix = lambda i: (0, 0)
    return pl.pallas_call(
        body,
        grid=(_N // _RB,),
        in_specs=[
            pl.BlockSpec((_RB, _DH), row),
            pl.BlockSpec((_RB, _DH), hi),
            pl.BlockSpec((1, _NS, _RB), lambda i: (i, 0, 0)),
            pl.BlockSpec((_RB, _D), row),
            pl.BlockSpec((dout, _D), fix),
            pl.BlockSpec((dout, _D), fix),
            pl.BlockSpec((1, dout), fix),
        ],
        out_specs=pl.BlockSpec((_RB, dout), row),
        out_shape=jax.ShapeDtypeStruct((_N, dout), F32),
    )


def kernel(x, edge_index, W1_l, b1_l, W1_r, W2_l, b2_l, W2_r):
    edges = edge_index.reshape(2, _NS, _NCHUNK, _CH)

    agg1, cnt = _sc_agg_cnt(x.reshape(_NC * _N, _DH), edges)

    h1 = _dense_call(_dense1_body, _D)(
        agg1, agg1, cnt, x, W1_l, W1_r, b1_l.reshape(1, _D))

    agg2 = _sc_agg(h1.reshape(_NC * _N, _DH), edges)

    return _dense_call(_dense2_body, 40)(
        agg2, agg2, cnt, h1, W2_l, W2_r, b2_l.reshape(1, 40))
